# baseline (speedup 1.0000x reference)
import functools

import jax
import jax.numpy as jnp
from jax import lax
from jax.experimental import pallas as pl
from jax.experimental.pallas import tpu as pltpu
from jax.experimental.pallas import tpu_sc as plsc

_L = 16
_D = 64
_SEQ = 200
_NCH = _D // _L
_NW = 32
_NACT = 25
_KG = 8
_W = 128
_SLAB = 128
_NBUF = 4


def _gather_body(idx_hbm, embt_hbm, part_hbm, idx_v, bufs_v, acc_v,
                 sem0, sem1, sem2, sem3):
    c = lax.axis_index("c")
    s = lax.axis_index("s")
    wid = s * 2 + c
    sems = (sem0, sem1, sem2, sem3)

    @pl.when(wid < _NACT)
    def _():
        _gather_tile(idx_hbm, embt_hbm, part_hbm, idx_v, bufs_v, acc_v,
                     sems, wid)


def _gather_tile(idx_hbm, embt_hbm, part_hbm, idx_v, bufs_v, acc_v,
                 sems, wid):
    pltpu.sync_copy(idx_hbm.at[pl.ds(wid * _KG, _KG)],
                    idx_v.at[pl.ds(0, _KG)])
    chunk = idx_v[...]

    starts = []
    lanes = []
    for i in range(_KG):
        r = chunk[i]
        st = (r // _W) * _W
        starts.append(st)
        lanes.append(r - st)

    def issue(i):
        return pltpu.async_copy(
            embt_hbm.at[:, pl.ds(starts[i], _W)],
            bufs_v.at[i % _NBUF], sems[i % _NBUF])

    cps = [issue(i) for i in range(_NBUF - 1)]
    accs = [None] * _NCH
    for i in range(_KG):
        if i + _NBUF - 1 < _KG:
            cps.append(issue(i + _NBUF - 1))
        cps[i].wait()
        buf = bufs_v.at[i % _NBUF]
        col = jnp.full((_L,), lanes[i], jnp.int32)
        for d in range(_NCH):
            row = lax.iota(jnp.int32, _L) + (d * _L)
            v = plsc.load_gather(buf, [row, col])
            accs[d] = v if accs[d] is None else jnp.maximum(accs[d], v)

    for d in range(_NCH):
        acc_v[pl.ds(d * _L, _L)] = accs[d]
    pltpu.sync_copy(acc_v, part_hbm.at[pl.ds(wid * _SLAB, _SLAB)])


def _reduce_body(part_hbm, w_hbm, b_hbm, out_hbm, part_v, w_v, b_v, out_v):
    c = lax.axis_index("c")
    s = lax.axis_index("s")

    @pl.when(jnp.logical_and(c == 0, s == 0))
    def _():
        pltpu.sync_copy(part_hbm, part_v)
        pltpu.sync_copy(w_hbm, w_v)
        pltpu.sync_copy(b_hbm, b_v.at[pl.ds(0, 2)])

        pooled = []
        for d in range(_NCH):
            m = part_v[pl.ds(d * _L, _L)]
            for t in range(1, _NACT):
                m = jnp.maximum(m, part_v[pl.ds(t * _SLAB + d * _L, _L)])
            pooled.append(m)

        lane = lax.iota(jnp.int32, _L)
        vec = jnp.zeros((_L,), jnp.float32)
        for j in range(2):
            psum = jnp.zeros((_L,), jnp.float32)
            for d in range(_NCH):
                psum = psum + pooled[d] * w_v[j, pl.ds(d * _L, _L)]
            t = psum[0]
            for i in range(1, _L):
                t = t + psum[i]
            vec = jnp.where(lane == j, t, vec)
        out_v[...] = vec + b_v[...]
        pltpu.sync_copy(out_v.at[pl.ds(0, 2)], out_hbm.at[0])


_mesh = plsc.VectorSubcoreMesh(core_axis_name="c", subcore_axis_name="s",
                               num_cores=2, num_subcores=16)
_mesh1 = plsc.VectorSubcoreMesh(core_axis_name="c", subcore_axis_name="s",
                                num_cores=1, num_subcores=16)

_gather = functools.partial(
    pl.kernel,
    out_type=jax.ShapeDtypeStruct((_NACT * _SLAB,), jnp.float32),
    mesh=_mesh,
    compiler_params=pltpu.CompilerParams(needs_layout_passes=False, skip_device_barrier=True),
    scratch_types=[
        pltpu.VMEM((_L,), jnp.int32),
        pltpu.VMEM((_NBUF, _D, _W), jnp.float32),
        pltpu.VMEM((_SLAB,), jnp.float32),
        pltpu.SemaphoreType.DMA,
        pltpu.SemaphoreType.DMA,
        pltpu.SemaphoreType.DMA,
        pltpu.SemaphoreType.DMA,
    ],
)(_gather_body)

_reduce = functools.partial(
    pl.kernel,
    out_type=jax.ShapeDtypeStruct((1, 2), jnp.float32),
    mesh=_mesh1,
    compiler_params=pltpu.CompilerParams(needs_layout_passes=False, skip_device_barrier=True),
    scratch_types=[
        pltpu.VMEM((_NACT * _SLAB,), jnp.float32),
        pltpu.VMEM((2, _D), jnp.float32),
        pltpu.VMEM((_L,), jnp.float32),
        pltpu.VMEM((_L,), jnp.float32),
    ],
)(_reduce_body)


@jax.jit
def kernel(x, emb, W, b):
    idx = x.reshape(-1).astype(jnp.int32)
    part = _gather(idx, emb.T)
    return _reduce(part, W.astype(jnp.float32), b.astype(jnp.float32))


# speedup vs baseline: 1.0237x; 1.0237x over previous
import functools

import jax
import jax.numpy as jnp
from jax import lax
from jax.experimental import pallas as pl
from jax.experimental.pallas import tpu as pltpu
from jax.experimental.pallas import tpu_sc as plsc

_L = 16
_D = 64
_SEQ = 200
_NCH = _D // _L
_NW = 32
_NACT = 25
_KG = 8
_W = 128
_SLAB = 128
_NBUF = 6


def _gather_body(idx_hbm, embt_hbm, part_hbm, idx_v, bufs_v, acc_v,
                 sem0, sem1, sem2, sem3, sem4, sem5):
    c = lax.axis_index("c")
    s = lax.axis_index("s")
    wid = s * 2 + c
    sems = (sem0, sem1, sem2, sem3, sem4, sem5)

    @pl.when(wid < _NACT)
    def _():
        _gather_tile(idx_hbm, embt_hbm, part_hbm, idx_v, bufs_v, acc_v,
                     sems, wid)


def _gather_tile(idx_hbm, embt_hbm, part_hbm, idx_v, bufs_v, acc_v,
                 sems, wid):
    pltpu.sync_copy(idx_hbm.at[pl.ds(wid * _KG, _KG)],
                    idx_v.at[pl.ds(0, _KG)])
    chunk = idx_v[...]

    starts = []
    lanes = []
    for i in range(_KG):
        r = chunk[i]
        st = (r // _W) * _W
        starts.append(st)
        lanes.append(r - st)

    def issue(i):
        return pltpu.async_copy(
            embt_hbm.at[:, pl.ds(starts[i], _W)],
            bufs_v.at[i % _NBUF], sems[i % _NBUF])

    cps = [issue(i) for i in range(_NBUF - 1)]
    accs = [None] * _NCH
    for i in range(_KG):
        if i + _NBUF - 1 < _KG:
            cps.append(issue(i + _NBUF - 1))
        cps[i].wait()
        buf = bufs_v.at[i % _NBUF]
        col = jnp.full((_L,), lanes[i], jnp.int32)
        for d in range(_NCH):
            row = lax.iota(jnp.int32, _L) + (d * _L)
            v = plsc.load_gather(buf, [row, col])
            accs[d] = v if accs[d] is None else jnp.maximum(accs[d], v)

    for d in range(_NCH):
        acc_v[pl.ds(d * _L, _L)] = accs[d]
    pltpu.sync_copy(acc_v, part_hbm.at[pl.ds(wid * _SLAB, _SLAB)])


def _reduce_body(part_hbm, w_hbm, b_hbm, out_hbm, part_v, w_v, b_v, out_v):
    c = lax.axis_index("c")
    s = lax.axis_index("s")

    @pl.when(jnp.logical_and(c == 0, s == 0))
    def _():
        pltpu.sync_copy(part_hbm, part_v)
        pltpu.sync_copy(w_hbm, w_v)
        pltpu.sync_copy(b_hbm, b_v.at[pl.ds(0, 2)])

        pooled = []
        for d in range(_NCH):
            vals = [part_v[pl.ds(t * _SLAB + d * _L, _L)]
                    for t in range(_NACT)]
            while len(vals) > 1:
                nxt = [jnp.maximum(vals[k], vals[k + 1])
                       for k in range(0, len(vals) - 1, 2)]
                if len(vals) % 2:
                    nxt.append(vals[-1])
                vals = nxt
            pooled.append(vals[0])

        lane = lax.iota(jnp.int32, _L)
        vec = jnp.zeros((_L,), jnp.float32)
        for j in range(2):
            psum = jnp.zeros((_L,), jnp.float32)
            for d in range(_NCH):
                psum = psum + pooled[d] * w_v[j, pl.ds(d * _L, _L)]
            t = psum[0]
            for i in range(1, _L):
                t = t + psum[i]
            vec = jnp.where(lane == j, t, vec)
        out_v[...] = vec + b_v[...]
        pltpu.sync_copy(out_v.at[pl.ds(0, 2)], out_hbm.at[0])


_mesh = plsc.VectorSubcoreMesh(core_axis_name="c", subcore_axis_name="s",
                               num_cores=2, num_subcores=16)
_mesh1 = plsc.VectorSubcoreMesh(core_axis_name="c", subcore_axis_name="s",
                                num_cores=1, num_subcores=16)

_gather = functools.partial(
    pl.kernel,
    out_type=jax.ShapeDtypeStruct((_NACT * _SLAB,), jnp.float32),
    mesh=_mesh,
    compiler_params=pltpu.CompilerParams(needs_layout_passes=False, skip_device_barrier=True),
    scratch_types=[
        pltpu.VMEM((_L,), jnp.int32),
        pltpu.VMEM((_NBUF, _D, _W), jnp.float32),
        pltpu.VMEM((_SLAB,), jnp.float32),
        pltpu.SemaphoreType.DMA,
        pltpu.SemaphoreType.DMA,
        pltpu.SemaphoreType.DMA,
        pltpu.SemaphoreType.DMA,
        pltpu.SemaphoreType.DMA,
        pltpu.SemaphoreType.DMA,
    ],
)(_gather_body)

_reduce = functools.partial(
    pl.kernel,
    out_type=jax.ShapeDtypeStruct((1, 2), jnp.float32),
    mesh=_mesh1,
    compiler_params=pltpu.CompilerParams(needs_layout_passes=False, skip_device_barrier=True),
    scratch_types=[
        pltpu.VMEM((_NACT * _SLAB,), jnp.float32),
        pltpu.VMEM((2, _D), jnp.float32),
        pltpu.VMEM((_L,), jnp.float32),
        pltpu.VMEM((_L,), jnp.float32),
    ],
)(_reduce_body)


@jax.jit
def kernel(x, emb, W, b):
    idx = x.reshape(-1).astype(jnp.int32)
    part = _gather(idx, emb.T)
    return _reduce(part, W.astype(jnp.float32), b.astype(jnp.float32))
